# Initial kernel scaffold; baseline (speedup 1.0000x reference)
import jax
import jax.numpy as jnp
from jax.experimental import pallas as pl


def kernel(input, embedding_weight, fusion_weight):
    raise NotImplementedError("write your pallas kernel here")


# speedup vs baseline: 7.4397x; 7.4397x over previous
import functools

import jax
import jax.numpy as jnp
from jax import lax
from jax.experimental import pallas as pl
from jax.experimental.pallas import tpu as pltpu
from jax.experimental.pallas import tpu_sc as plsc

V = 100000
A = 1024
D = 64
B = 4096
S = 200
NTOK = B * S
NC = 2
NS = 16
NW = NC * NS
TOK_PER_W = NTOK // NW

IDX_W = 128
SUB = 8
CHUNK = IDX_W * SUB
N_CHUNK = TOK_PER_W // CHUNK

MAIN_CHUNK = 800
MAIN_NCH = V // MAIN_CHUNK
FUS_PER_W = A // NW

_mesh = plsc.VectorSubcoreMesh(core_axis_name="c", subcore_axis_name="s")


def _wid():
    return lax.axis_index("s") * NC + lax.axis_index("c")


def _concat_body(main_hbm, fus_hbm, out_hbm, buf, sem):
    w = _wid()
    for t in range(-(-MAIN_NCH // NW)):
        c = t * NW + w

        @pl.when(c < MAIN_NCH)
        def _():
            r0 = pl.multiple_of(c * MAIN_CHUNK, 8)
            pltpu.sync_copy(main_hbm.at[pl.ds(r0, MAIN_CHUNK)], buf)
            pltpu.sync_copy(buf, out_hbm.at[pl.ds(r0, MAIN_CHUNK)])
    f0 = pl.multiple_of(w * FUS_PER_W, 8)
    pltpu.sync_copy(fus_hbm.at[pl.ds(f0, FUS_PER_W)], buf.at[pl.ds(0, FUS_PER_W)])
    pltpu.sync_copy(
        buf.at[pl.ds(0, FUS_PER_W)],
        out_hbm.at[pl.ds(pl.multiple_of(V + w * FUS_PER_W, 8), FUS_PER_W)],
    )


@functools.partial(
    pl.kernel,
    mesh=_mesh,
    out_type=jax.ShapeDtypeStruct((V + A, D), jnp.float32),
    scratch_types=[
        pltpu.VMEM((MAIN_CHUNK, D), jnp.float32),
        pltpu.SemaphoreType.DMA,
    ],
)
def _build_concat(main_hbm, fus_hbm, out_hbm, buf, sem):
    _concat_body(main_hbm, fus_hbm, out_hbm, buf, sem)


@functools.partial(
    pl.kernel,
    mesh=_mesh,
    out_type=jax.ShapeDtypeStruct((NTOK, D), jnp.float32),
    compiler_params=pltpu.CompilerParams(use_tc_tiling_on_sc=False),
    scratch_types=[
        pltpu.VMEM((SUB, IDX_W), jnp.int32),
        pltpu.VMEM((CHUNK, D), jnp.float32),
        pltpu.SemaphoreType.DMA,
    ],
)
def _gather(table_hbm, idx_hbm, out_hbm, idx_v, rows_v, sem):
    w = _wid()
    base = w * TOK_PER_W

    def body(g, _):
        t0 = pl.multiple_of(base + g * CHUNK, CHUNK)
        pltpu.sync_copy(idx_hbm.at[pl.ds(pl.multiple_of(t0 // IDX_W, 8), SUB)], idx_v)
        copies = [
            pltpu.async_copy(
                table_hbm.at[idx_v.at[j]],
                rows_v.at[pl.ds(j * IDX_W, IDX_W)],
                sem,
            )
            for j in range(SUB)
        ]
        for c in copies:
            c.wait()
        pltpu.sync_copy(rows_v, out_hbm.at[pl.ds(t0, CHUNK)])
        return _

    lax.fori_loop(0, N_CHUNK, body, None)


def kernel(input, embedding_weight, fusion_weight):
    idx = input.reshape(NTOK // IDX_W, IDX_W).astype(jnp.int32)
    table = _build_concat(embedding_weight, fusion_weight)
    out = _gather(table, idx)
    return out.reshape(B, S, D)


# speedup vs baseline: 7.6223x; 1.0245x over previous
import functools

import jax
import jax.numpy as jnp
from jax import lax
from jax.experimental import pallas as pl
from jax.experimental.pallas import tpu as pltpu
from jax.experimental.pallas import tpu_sc as plsc

V = 100000
A = 1024
D = 64
B = 4096
S = 200
NTOK = B * S
NC = 2
NS = 16
NW = NC * NS
TOK_PER_W = NTOK // NW

IDX_W = 128
IDX_ROWS_W = TOK_PER_W // IDX_W
UNIT = 640
GPU_ = UNIT // IDX_W
N_UNIT = TOK_PER_W // UNIT

FUS_PER_W = A // NW
MAIN_CHUNK = 400
MAIN_NCH = V // MAIN_CHUNK

_mesh = plsc.VectorSubcoreMesh(core_axis_name="c", subcore_axis_name="s")


def _wid():
    return lax.axis_index("s") * NC + lax.axis_index("c")


@functools.partial(
    pl.kernel,
    mesh=_mesh,
    out_type=jax.ShapeDtypeStruct((V + A, D), jnp.float32),
    scratch_types=[
        pltpu.VMEM((2, MAIN_CHUNK, D), jnp.float32),
        pltpu.SemaphoreType.DMA,
        pltpu.SemaphoreType.DMA,
        pltpu.SemaphoreType.DMA,
        pltpu.SemaphoreType.DMA,
    ],
)
def _build_concat(main_hbm, fus_hbm, out_hbm, buf, lsem0, lsem1, ssem0, ssem1):
    w = _wid()
    lsems = (lsem0, lsem1)
    ssems = (ssem0, ssem1)
    loads = [None, None]
    stores = [None, None]
    for t in range(7):
        b = t % 2
        r0 = pl.multiple_of((t * NW + w) * MAIN_CHUNK, 8)
        if stores[b] is not None:
            stores[b].wait()
        loads[b] = pltpu.async_copy(main_hbm.at[pl.ds(r0, MAIN_CHUNK)], buf.at[b], lsems[b])
        loads[b].wait()
        stores[b] = pltpu.async_copy(buf.at[b], out_hbm.at[pl.ds(r0, MAIN_CHUNK)], ssems[b])
    stores[0].wait()
    stores[1].wait()
    @pl.when(w < MAIN_NCH - 7 * NW)
    def _():
        r0 = pl.multiple_of((7 * NW + w) * MAIN_CHUNK, 8)
        pltpu.sync_copy(main_hbm.at[pl.ds(r0, MAIN_CHUNK)], buf.at[0].at[pl.ds(0, MAIN_CHUNK)])
        pltpu.sync_copy(buf.at[0].at[pl.ds(0, MAIN_CHUNK)], out_hbm.at[pl.ds(r0, MAIN_CHUNK)])
    f0 = pl.multiple_of(w * FUS_PER_W, 8)
    pltpu.sync_copy(fus_hbm.at[pl.ds(f0, FUS_PER_W)], buf.at[1].at[pl.ds(0, FUS_PER_W)])
    pltpu.sync_copy(
        buf.at[1].at[pl.ds(0, FUS_PER_W)],
        out_hbm.at[pl.ds(pl.multiple_of(V + w * FUS_PER_W, 8), FUS_PER_W)],
    )


@functools.partial(
    pl.kernel,
    mesh=_mesh,
    out_type=jax.ShapeDtypeStruct((NTOK, D), jnp.float32),
    compiler_params=pltpu.CompilerParams(use_tc_tiling_on_sc=False),
    scratch_types=[
        pltpu.VMEM((IDX_ROWS_W, IDX_W), jnp.int32),
        pltpu.VMEM((2, UNIT, D), jnp.float32),
        pltpu.SemaphoreType.DMA,
        pltpu.SemaphoreType.DMA,
        pltpu.SemaphoreType.DMA,
    ],
)
def _gather(table_hbm, idx_hbm, out_hbm, idx_v, rows_v, gsem, osem0, osem1):
    w = _wid()
    base = w * TOK_PER_W
    pltpu.sync_copy(idx_hbm.at[pl.ds(pl.multiple_of(w * IDX_ROWS_W, 8), IDX_ROWS_W)], idx_v)
    osems = (osem0, osem1)

    def pair(i, _):
        for b in range(2):
            u = 2 * i + b
            @pl.when(i >= 1)
            def _():
                pltpu.make_async_copy(
                    rows_v.at[b], out_hbm.at[pl.ds(0, UNIT)], osems[b]
                ).wait()

            copies = [
                pltpu.async_copy(
                    table_hbm.at[idx_v.at[u * GPU_ + j]],
                    rows_v.at[b].at[pl.ds(j * IDX_W, IDX_W)],
                    gsem,
                )
                for j in range(GPU_)
            ]
            for c in copies:
                c.wait()
            t0 = pl.multiple_of(base + u * UNIT, 8)
            pltpu.async_copy(rows_v.at[b], out_hbm.at[pl.ds(t0, UNIT)], osems[b])
        return _

    lax.fori_loop(0, N_UNIT // 2, pair, None)
    for b in range(2):
        pltpu.make_async_copy(rows_v.at[b], out_hbm.at[pl.ds(0, UNIT)], osems[b]).wait()


def kernel(input, embedding_weight, fusion_weight):
    idx = input.reshape(NTOK // IDX_W, IDX_W).astype(jnp.int32)
    table = _build_concat(embedding_weight, fusion_weight)
    out = _gather(table, idx)
    return out.reshape(B, S, D)
